# Initial kernel scaffold; baseline (speedup 1.0000x reference)
import jax
import jax.numpy as jnp
from jax.experimental import pallas as pl


def kernel(x, edge_index, W0, b0, W1, b1, W2, b2, g0, beta0, g1, beta1):
    raise NotImplementedError("write your pallas kernel here")


# speedup vs baseline: 53.1365x; 53.1365x over previous
import functools

import jax
import jax.numpy as jnp
from jax import lax
from jax.experimental import pallas as pl
from jax.experimental.pallas import tpu as pltpu
from jax.experimental.pallas import tpu_sc as plsc

N = 10000
E = 320000
NC = 2
NS = 16
NW = NC * NS
CHUNK = 128
CH = (E + NW * CHUNK - 1) // (NW * CHUNK)
EPAD = NW * CH * CHUNK
NP = 10240
RPT = NP // NS

_MESH = plsc.VectorSubcoreMesh(core_axis_name="c", subcore_axis_name="s")
_SC_PARAMS = pltpu.CompilerParams(needs_layout_passes=False,
                                  use_tc_tiling_on_sc=False)


def _zero_1d(ref, nwords):
    z16 = jnp.zeros((16,), jnp.float32)

    @pl.loop(0, nwords // 16)
    def _(i):
        ref[pl.ds(i * 16, 16)] = z16


def _zero_2d(ref, nrows, ncols):
    z16 = jnp.zeros((16,), jnp.float32)

    @pl.loop(0, nrows)
    def _(r):
        for k in range(ncols // 16):
            ref[r, pl.ds(k * 16, 16)] = z16


@functools.partial(
    pl.kernel,
    out_type=jax.ShapeDtypeStruct((NW, NP), jnp.float32),
    mesh=_MESH,
    compiler_params=_SC_PARAMS,
    scratch_types=[
        pltpu.VMEM((CH, CHUNK), jnp.int32),
        pltpu.VMEM((NP,), jnp.float32),
    ],
)
def _sc_degree(dst3, counts_out, dst_v, cnt_v):
    cid = lax.axis_index("c")
    sid = lax.axis_index("s")
    w = cid * NS + sid
    pltpu.sync_copy(dst3.at[w], dst_v)
    _zero_1d(cnt_v, NP)
    ones = jnp.ones((16,), jnp.float32)

    @pl.loop(0, CH)
    def _(j):
        for k in range(CHUNK // 16):
            idx = dst_v[j, pl.ds(k * 16, 16)]
            plsc.addupdate_scatter(cnt_v, [idx], ones)

    pltpu.sync_copy(cnt_v, counts_out.at[w])


@functools.partial(
    pl.kernel,
    out_type=jax.ShapeDtypeStruct((NC, NP, 32), jnp.float32),
    mesh=_MESH,
    compiler_params=_SC_PARAMS,
    scratch_types=[
        pltpu.VMEM((CH, CHUNK), jnp.int32),
        pltpu.VMEM((CH, CHUNK), jnp.int32),
        pltpu.VMEM((CHUNK, 32), jnp.float32),
        pltpu.VMEM((CHUNK, 32), jnp.float32),
        pltpu.VMEM((CHUNK, 32), jnp.float32),
        pltpu.VMEM((CHUNK, 32), jnp.float32),
        pltpu.VMEM((CHUNK, 32), jnp.float32),
        pltpu.VMEM_SHARED((NP, 32), jnp.float32),
        pltpu.VMEM_SHARED((N, 32), jnp.float32),
        pltpu.SemaphoreType.DMA,
        pltpu.SemaphoreType.DMA,
        pltpu.SemaphoreType.DMA,
        pltpu.SemaphoreType.DMA,
    ],
)
def _sc_prop32(src3, dst3, h_hbm, psum_out, src_v, dst_v, rows0, rows1,
               rows2, rows3, zrow_v, acc_sh, h_sh, sem0, sem1, sem2, sem3):
    rows = (rows0, rows1, rows2, rows3)
    sems = (sem0, sem1, sem2, sem3)
    rows_a = rows0
    cid = lax.axis_index("c")
    sid = lax.axis_index("s")
    w = cid * NS + sid
    pltpu.sync_copy(src3.at[w], src_v)
    pltpu.sync_copy(dst3.at[w], dst_v)
    pltpu.sync_copy(h_hbm.at[pl.ds(sid * (N // NS), N // NS)],
                    h_sh.at[pl.ds(sid * (N // NS), N // NS)])
    _zero_2d(zrow_v, CHUNK, 32)
    for i in range(RPT // CHUNK):
        pltpu.sync_copy(zrow_v, acc_sh.at[pl.ds(sid * RPT + i * CHUNK, CHUNK)])
    plsc.subcore_barrier()

    for b in range(4):
        pltpu.async_copy(h_sh.at[src_v.at[b]], rows[b], sems[b])

    @pl.loop(0, CH // 4 - 1)
    def _(g):
        j = 4 * g
        for b in range(4):
            pltpu.make_async_copy(
                h_sh.at[src_v.at[j + b]], rows[b], sems[b]).wait()
            pltpu.sync_copy(rows[b], acc_sh.at[dst_v.at[j + b]], add=True)
            pltpu.async_copy(h_sh.at[src_v.at[j + b + 4]], rows[b], sems[b])

    t0 = 4 * (CH // 4 - 1)
    for b in range(4):
        pltpu.make_async_copy(
            h_sh.at[src_v.at[t0 + b]], rows[b], sems[b]).wait()
        pltpu.sync_copy(rows[b], acc_sh.at[dst_v.at[t0 + b]], add=True)
        if t0 + b + 4 < CH:
            pltpu.async_copy(
                h_sh.at[src_v.at[t0 + b + 4]], rows[b], sems[b])
    for b in range(CH - t0 - 4):
        pltpu.make_async_copy(
            h_sh.at[src_v.at[t0 + 4 + b]], rows[b], sems[b]).wait()
        pltpu.sync_copy(rows[b], acc_sh.at[dst_v.at[t0 + 4 + b]], add=True)

    plsc.subcore_barrier()
    for i in range(RPT // CHUNK):
        r = sid * RPT + i * CHUNK
        pltpu.sync_copy(acc_sh.at[pl.ds(r, CHUNK)], rows_a)
        pltpu.sync_copy(rows_a, psum_out.at[cid, pl.ds(r, CHUNK)])


@functools.partial(
    pl.kernel,
    out_type=jax.ShapeDtypeStruct((NW, NP), jnp.float32),
    mesh=_MESH,
    compiler_params=_SC_PARAMS,
    scratch_types=[
        pltpu.VMEM((CH, CHUNK), jnp.int32),
        pltpu.VMEM((CH, CHUNK), jnp.int32),
        pltpu.VMEM((N,), jnp.float32),
        pltpu.VMEM((NP,), jnp.float32),
    ],
)
def _sc_prop1(src3, dst3, t_hbm, psum_out, src_v, dst_v, t_v, acc_v):
    cid = lax.axis_index("c")
    sid = lax.axis_index("s")
    w = cid * NS + sid
    pltpu.sync_copy(t_hbm, t_v)
    pltpu.sync_copy(src3.at[w], src_v)
    pltpu.sync_copy(dst3.at[w], dst_v)
    _zero_1d(acc_v, NP)

    @pl.loop(0, CH)
    def _(j):
        for k in range(CHUNK // 16):
            sidx = src_v[j, pl.ds(k * 16, 16)]
            didx = dst_v[j, pl.ds(k * 16, 16)]
            vals = plsc.load_gather(t_v, [sidx])
            plsc.addupdate_scatter(acc_v, [didx], vals)

    pltpu.sync_copy(acc_v, psum_out.at[w])


def _tc_a_body(ct_ref, x_ref, w0_ref, dinv_ref, h0s_ref):
    ct = ct_ref[...]
    deg = jnp.sum(ct, axis=1, keepdims=True) + 1.0
    dinv = lax.rsqrt(deg)
    dinv_ref[...] = dinv
    h0 = jnp.dot(x_ref[...], w0_ref[...], preferred_element_type=jnp.float32)
    h0s_ref[...] = h0 * dinv[:N]


def _tc_mid_body(p_ref, hs_ref, dinv_ref, b_ref, g_ref, beta_ref, w_ref,
                 out_ref):
    dinv = dinv_ref[...][:N]
    p = p_ref[...]
    z = dinv * (p[0, :N] + p[1, :N] + hs_ref[...]) + b_ref[...]
    mu = jnp.mean(z, axis=0, keepdims=True)
    zc = z - mu
    var = jnp.mean(zc * zc, axis=0, keepdims=True)
    zn = g_ref[...] * zc * lax.rsqrt(var + 1e-5) + beta_ref[...]
    a = jnp.maximum(zn, 0.0)
    out_ref[...] = (
        jnp.dot(a, w_ref[...], preferred_element_type=jnp.float32) * dinv
    )


def _tc_d_body(p2_ref, h2s_ref, dinv_ref, b2_ref, out_ref):
    dinv = dinv_ref[...][:N]
    p2 = p2_ref[...]
    psum = jnp.sum(p2[:N], axis=1, keepdims=True)
    z = dinv * (psum + h2s_ref[...]) + b2_ref[...]
    out_ref[...] = jax.nn.sigmoid(z)


def kernel(x, edge_index, W0, b0, W1, b1, W2, b2, g0, beta0, g1, beta1):
    ei = edge_index.astype(jnp.int32)
    pad = EPAD - E
    src3 = jnp.concatenate([ei[0], jnp.zeros((pad,), jnp.int32)])
    src3 = src3.reshape(NW, CH, CHUNK)
    dst3 = jnp.concatenate([ei[1], jnp.full((pad,), N, jnp.int32)])
    dst3 = dst3.reshape(NW, CH, CHUNK)

    counts = _sc_degree(dst3)

    dinv_col, h0s = pl.pallas_call(
        _tc_a_body,
        out_shape=[
            jax.ShapeDtypeStruct((NP, 1), jnp.float32),
            jax.ShapeDtypeStruct((N, 32), jnp.float32),
        ],
    )(counts.T, x, W0)

    p0 = _sc_prop32(src3, dst3, h0s)
    h1s = pl.pallas_call(
        _tc_mid_body,
        out_shape=jax.ShapeDtypeStruct((N, 32), jnp.float32),
    )(p0, h0s, dinv_col, b0.reshape(1, 32), g0.reshape(1, 32),
      beta0.reshape(1, 32), W1)

    p1 = _sc_prop32(src3, dst3, h1s)
    h2s = pl.pallas_call(
        _tc_mid_body,
        out_shape=jax.ShapeDtypeStruct((N, 1), jnp.float32),
    )(p1, h1s, dinv_col, b1.reshape(1, 32), g1.reshape(1, 32),
      beta1.reshape(1, 32), W2)

    p2 = _sc_prop1(src3, dst3, h2s.reshape(N))
    out = pl.pallas_call(
        _tc_d_body,
        out_shape=jax.ShapeDtypeStruct((N, 1), jnp.float32),
    )(p2.T, h2s, dinv_col, b2.reshape(1, 1))
    return out


# speedup vs baseline: 54.9737x; 1.0346x over previous
import functools

import jax
import jax.numpy as jnp
from jax import lax
from jax.experimental import pallas as pl
from jax.experimental.pallas import tpu as pltpu
from jax.experimental.pallas import tpu_sc as plsc

N = 10000
E = 320000
NC = 2
NS = 16
NW = NC * NS
CHUNK = 128
CH = (E + NW * CHUNK - 1) // (NW * CHUNK)
EPAD = NW * CH * CHUNK
NP = 10240
RPT = NP // NS

_MESH = plsc.VectorSubcoreMesh(core_axis_name="c", subcore_axis_name="s")
_SC_PARAMS = pltpu.CompilerParams(needs_layout_passes=False,
                                  use_tc_tiling_on_sc=False)


def _zero_1d(ref, nwords):
    z16 = jnp.zeros((16,), jnp.float32)

    @pl.loop(0, nwords // 16)
    def _(i):
        ref[pl.ds(i * 16, 16)] = z16


def _zero_2d(ref, nrows, ncols):
    z16 = jnp.zeros((16,), jnp.float32)

    @pl.loop(0, nrows)
    def _(r):
        for k in range(ncols // 16):
            ref[r, pl.ds(k * 16, 16)] = z16


@functools.partial(
    pl.kernel,
    out_type=jax.ShapeDtypeStruct((NW, NP), jnp.float32),
    mesh=_MESH,
    compiler_params=_SC_PARAMS,
    scratch_types=[
        pltpu.VMEM((CH, CHUNK), jnp.int32),
        pltpu.VMEM((NP,), jnp.float32),
    ],
)
def _sc_degree(dst3, counts_out, dst_v, cnt_v):
    cid = lax.axis_index("c")
    sid = lax.axis_index("s")
    w = cid * NS + sid
    pltpu.sync_copy(dst3.at[w], dst_v)
    _zero_1d(cnt_v, NP)
    ones = jnp.ones((16,), jnp.float32)

    @pl.loop(0, CH)
    def _(j):
        for k in range(CHUNK // 16):
            idx = dst_v[j, pl.ds(k * 16, 16)]
            plsc.addupdate_scatter(cnt_v, [idx], ones)

    pltpu.sync_copy(cnt_v, counts_out.at[w])


@functools.partial(
    pl.kernel,
    out_type=jax.ShapeDtypeStruct((NC, NP, 32), jnp.float32),
    mesh=_MESH,
    compiler_params=_SC_PARAMS,
    scratch_types=[
        pltpu.VMEM((CH, CHUNK), jnp.int32),
        pltpu.VMEM((CH, CHUNK), jnp.int32),
        pltpu.VMEM((CHUNK, 32), jnp.float32),
        pltpu.VMEM((CHUNK, 32), jnp.float32),
        pltpu.VMEM((CHUNK, 32), jnp.float32),
        pltpu.VMEM((CHUNK, 32), jnp.float32),
        pltpu.VMEM((CHUNK, 32), jnp.float32),
        pltpu.VMEM_SHARED((NP, 32), jnp.float32),
        pltpu.VMEM_SHARED((N, 32), jnp.float32),
        pltpu.SemaphoreType.DMA,
        pltpu.SemaphoreType.DMA,
        pltpu.SemaphoreType.DMA,
        pltpu.SemaphoreType.DMA,
    ],
)
def _sc_prop32(src3, dst3, h_hbm, psum_out, src_v, dst_v, rows0, rows1,
               rows2, rows3, zrow_v, acc_sh, h_sh, sem0, sem1, sem2, sem3):
    rows = (rows0, rows1, rows2, rows3)
    sems = (sem0, sem1, sem2, sem3)
    rows_a = rows0
    cid = lax.axis_index("c")
    sid = lax.axis_index("s")
    w = cid * NS + sid
    pltpu.sync_copy(src3.at[w], src_v)
    pltpu.sync_copy(dst3.at[w], dst_v)
    pltpu.sync_copy(h_hbm.at[pl.ds(sid * (N // NS), N // NS)],
                    h_sh.at[pl.ds(sid * (N // NS), N // NS)])
    _zero_2d(zrow_v, CHUNK, 32)
    for i in range(RPT // CHUNK):
        pltpu.sync_copy(zrow_v, acc_sh.at[pl.ds(sid * RPT + i * CHUNK, CHUNK)])
    plsc.subcore_barrier()

    for b in range(4):
        pltpu.async_copy(h_sh.at[src_v.at[b]], rows[b], sems[b])

    @pl.loop(0, CH // 4 - 1)
    def _(g):
        j = 4 * g
        for b in range(4):
            pltpu.make_async_copy(
                h_sh.at[src_v.at[j + b]], rows[b], sems[b]).wait()
            pltpu.sync_copy(rows[b], acc_sh.at[dst_v.at[j + b]], add=True)
            pltpu.async_copy(h_sh.at[src_v.at[j + b + 4]], rows[b], sems[b])

    t0 = 4 * (CH // 4 - 1)
    for b in range(4):
        pltpu.make_async_copy(
            h_sh.at[src_v.at[t0 + b]], rows[b], sems[b]).wait()
        pltpu.sync_copy(rows[b], acc_sh.at[dst_v.at[t0 + b]], add=True)
        if t0 + b + 4 < CH:
            pltpu.async_copy(
                h_sh.at[src_v.at[t0 + b + 4]], rows[b], sems[b])
    for b in range(CH - t0 - 4):
        pltpu.make_async_copy(
            h_sh.at[src_v.at[t0 + 4 + b]], rows[b], sems[b]).wait()
        pltpu.sync_copy(rows[b], acc_sh.at[dst_v.at[t0 + 4 + b]], add=True)

    plsc.subcore_barrier()
    for i in range(RPT // CHUNK):
        r = sid * RPT + i * CHUNK
        pltpu.sync_copy(acc_sh.at[pl.ds(r, CHUNK)], rows_a)
        pltpu.sync_copy(rows_a, psum_out.at[cid, pl.ds(r, CHUNK)])


@functools.partial(
    pl.kernel,
    out_type=jax.ShapeDtypeStruct((NW, NP), jnp.float32),
    mesh=_MESH,
    compiler_params=_SC_PARAMS,
    scratch_types=[
        pltpu.VMEM((CH, CHUNK), jnp.int32),
        pltpu.VMEM((CH, CHUNK), jnp.int32),
        pltpu.VMEM((N,), jnp.float32),
        pltpu.VMEM((NP,), jnp.float32),
    ],
)
def _sc_prop1(src3, dst3, t_hbm, psum_out, src_v, dst_v, t_v, acc_v):
    cid = lax.axis_index("c")
    sid = lax.axis_index("s")
    w = cid * NS + sid
    pltpu.sync_copy(t_hbm, t_v)
    pltpu.sync_copy(src3.at[w], src_v)
    pltpu.sync_copy(dst3.at[w], dst_v)
    _zero_1d(acc_v, NP)

    @pl.loop(0, CH)
    def _(j):
        for k in range(CHUNK // 16):
            sidx = src_v[j, pl.ds(k * 16, 16)]
            didx = dst_v[j, pl.ds(k * 16, 16)]
            vals = plsc.load_gather(t_v, [sidx])
            plsc.addupdate_scatter(acc_v, [didx], vals)

    pltpu.sync_copy(acc_v, psum_out.at[w])


def _tc_a_body(ct_ref, x_ref, w0_ref, dinv_ref, h0s_ref):
    ct = ct_ref[...]
    deg = jnp.sum(ct, axis=0, keepdims=True) + 1.0
    dinv = jnp.transpose(lax.rsqrt(deg))
    dinv_ref[...] = dinv
    h0 = jnp.dot(x_ref[...], w0_ref[...], preferred_element_type=jnp.float32)
    h0s_ref[...] = h0 * dinv[:N]


def _tc_mid_body(p_ref, hs_ref, dinv_ref, b_ref, g_ref, beta_ref, w_ref,
                 out_ref):
    dinv = dinv_ref[...][:N]
    p = p_ref[...]
    z = dinv * (p[0, :N] + p[1, :N] + hs_ref[...]) + b_ref[...]
    mu = jnp.mean(z, axis=0, keepdims=True)
    zc = z - mu
    var = jnp.mean(zc * zc, axis=0, keepdims=True)
    zn = g_ref[...] * zc * lax.rsqrt(var + 1e-5) + beta_ref[...]
    a = jnp.maximum(zn, 0.0)
    out_ref[...] = (
        jnp.dot(a, w_ref[...], preferred_element_type=jnp.float32) * dinv
    )


def _tc_d_body(p2_ref, h2s_ref, dinv_ref, b2_ref, out_ref):
    dinv = dinv_ref[...][:N]
    p2 = p2_ref[...]
    psum = jnp.transpose(jnp.sum(p2, axis=0, keepdims=True))[:N]
    z = dinv * (psum + h2s_ref[...]) + b2_ref[...]
    out_ref[...] = jax.nn.sigmoid(z)


def kernel(x, edge_index, W0, b0, W1, b1, W2, b2, g0, beta0, g1, beta1):
    ei = edge_index.astype(jnp.int32)
    pad = EPAD - E
    src3 = jnp.concatenate([ei[0], jnp.zeros((pad,), jnp.int32)])
    src3 = src3.reshape(NW, CH, CHUNK)
    dst3 = jnp.concatenate([ei[1], jnp.full((pad,), N, jnp.int32)])
    dst3 = dst3.reshape(NW, CH, CHUNK)

    counts = _sc_degree(dst3)

    dinv_col, h0s = pl.pallas_call(
        _tc_a_body,
        out_shape=[
            jax.ShapeDtypeStruct((NP, 1), jnp.float32),
            jax.ShapeDtypeStruct((N, 32), jnp.float32),
        ],
    )(counts, x, W0)

    p0 = _sc_prop32(src3, dst3, h0s)
    h1s = pl.pallas_call(
        _tc_mid_body,
        out_shape=jax.ShapeDtypeStruct((N, 32), jnp.float32),
    )(p0, h0s, dinv_col, b0.reshape(1, 32), g0.reshape(1, 32),
      beta0.reshape(1, 32), W1)

    p1 = _sc_prop32(src3, dst3, h1s)
    h2s = pl.pallas_call(
        _tc_mid_body,
        out_shape=jax.ShapeDtypeStruct((N, 1), jnp.float32),
    )(p1, h1s, dinv_col, b1.reshape(1, 32), g1.reshape(1, 32),
      beta1.reshape(1, 32), W2)

    p2 = _sc_prop1(src3, dst3, h2s.reshape(N))
    out = pl.pallas_call(
        _tc_d_body,
        out_shape=jax.ShapeDtypeStruct((N, 1), jnp.float32),
    )(p2, h2s, dinv_col, b2.reshape(1, 1))
    return out


# speedup vs baseline: 54.9866x; 1.0002x over previous
import functools

import jax
import jax.numpy as jnp
from jax import lax
from jax.experimental import pallas as pl
from jax.experimental.pallas import tpu as pltpu
from jax.experimental.pallas import tpu_sc as plsc

N = 10000
E = 320000
NC = 2
NS = 16
NW = NC * NS
CHUNK = 128
CH = (E + NW * CHUNK - 1) // (NW * CHUNK)
EPAD = NW * CH * CHUNK
NP = 10240
RPT = NP // NS

_MESH = plsc.VectorSubcoreMesh(core_axis_name="c", subcore_axis_name="s")
_SC_PARAMS = pltpu.CompilerParams(needs_layout_passes=False,
                                  use_tc_tiling_on_sc=False)


def _zero_1d(ref, nwords):
    z16 = jnp.zeros((16,), jnp.float32)

    @pl.loop(0, nwords // 16)
    def _(i):
        ref[pl.ds(i * 16, 16)] = z16


def _zero_2d(ref, nrows, ncols):
    z16 = jnp.zeros((16,), jnp.float32)

    @pl.loop(0, nrows)
    def _(r):
        for k in range(ncols // 16):
            ref[r, pl.ds(k * 16, 16)] = z16


@functools.partial(
    pl.kernel,
    out_type=jax.ShapeDtypeStruct((NW, NP), jnp.float32),
    mesh=_MESH,
    compiler_params=_SC_PARAMS,
    scratch_types=[
        pltpu.VMEM((CH, CHUNK), jnp.int32),
        pltpu.VMEM((NP,), jnp.float32),
    ],
)
def _sc_degree(dst3, counts_out, dst_v, cnt_v):
    cid = lax.axis_index("c")
    sid = lax.axis_index("s")
    w = cid * NS + sid
    pltpu.sync_copy(dst3.at[pl.ds(w * CH, CH)], dst_v)
    _zero_1d(cnt_v, NP)
    ones = jnp.ones((16,), jnp.float32)

    @pl.loop(0, CH)
    def _(j):
        for k in range(CHUNK // 16):
            idx = dst_v[j, pl.ds(k * 16, 16)]
            plsc.addupdate_scatter(cnt_v, [idx], ones)

    pltpu.sync_copy(cnt_v, counts_out.at[w])


@functools.partial(
    pl.kernel,
    out_type=jax.ShapeDtypeStruct((NC, NP, 32), jnp.float32),
    mesh=_MESH,
    compiler_params=_SC_PARAMS,
    scratch_types=[
        pltpu.VMEM((CH, CHUNK), jnp.int32),
        pltpu.VMEM((CH, CHUNK), jnp.int32),
        pltpu.VMEM((CHUNK, 32), jnp.float32),
        pltpu.VMEM((CHUNK, 32), jnp.float32),
        pltpu.VMEM((CHUNK, 32), jnp.float32),
        pltpu.VMEM((CHUNK, 32), jnp.float32),
        pltpu.VMEM((CHUNK, 32), jnp.float32),
        pltpu.VMEM_SHARED((NP, 32), jnp.float32),
        pltpu.VMEM_SHARED((N, 32), jnp.float32),
        pltpu.SemaphoreType.DMA,
        pltpu.SemaphoreType.DMA,
        pltpu.SemaphoreType.DMA,
        pltpu.SemaphoreType.DMA,
    ],
)
def _sc_prop32(src3, dst3, h_hbm, psum_out, src_v, dst_v, rows0, rows1,
               rows2, rows3, zrow_v, acc_sh, h_sh, sem0, sem1, sem2, sem3):
    rows = (rows0, rows1, rows2, rows3)
    sems = (sem0, sem1, sem2, sem3)
    rows_a = rows0
    cid = lax.axis_index("c")
    sid = lax.axis_index("s")
    w = cid * NS + sid
    pltpu.sync_copy(src3.at[pl.ds(w * CH, CH)], src_v)
    pltpu.sync_copy(dst3.at[pl.ds(w * CH, CH)], dst_v)
    pltpu.sync_copy(h_hbm.at[pl.ds(sid * (N // NS), N // NS)],
                    h_sh.at[pl.ds(sid * (N // NS), N // NS)])
    _zero_2d(zrow_v, CHUNK, 32)
    for i in range(RPT // CHUNK):
        pltpu.sync_copy(zrow_v, acc_sh.at[pl.ds(sid * RPT + i * CHUNK, CHUNK)])
    plsc.subcore_barrier()

    for b in range(4):
        pltpu.async_copy(h_sh.at[src_v.at[b]], rows[b], sems[b])

    @pl.loop(0, CH // 4 - 1)
    def _(g):
        j = 4 * g
        for b in range(4):
            pltpu.make_async_copy(
                h_sh.at[src_v.at[j + b]], rows[b], sems[b]).wait()
            pltpu.sync_copy(rows[b], acc_sh.at[dst_v.at[j + b]], add=True)
            pltpu.async_copy(h_sh.at[src_v.at[j + b + 4]], rows[b], sems[b])

    t0 = 4 * (CH // 4 - 1)
    for b in range(4):
        pltpu.make_async_copy(
            h_sh.at[src_v.at[t0 + b]], rows[b], sems[b]).wait()
        pltpu.sync_copy(rows[b], acc_sh.at[dst_v.at[t0 + b]], add=True)
        if t0 + b + 4 < CH:
            pltpu.async_copy(
                h_sh.at[src_v.at[t0 + b + 4]], rows[b], sems[b])
    for b in range(CH - t0 - 4):
        pltpu.make_async_copy(
            h_sh.at[src_v.at[t0 + 4 + b]], rows[b], sems[b]).wait()
        pltpu.sync_copy(rows[b], acc_sh.at[dst_v.at[t0 + 4 + b]], add=True)

    plsc.subcore_barrier()
    for i in range(RPT // CHUNK):
        r = sid * RPT + i * CHUNK
        pltpu.sync_copy(acc_sh.at[pl.ds(r, CHUNK)], rows_a)
        pltpu.sync_copy(rows_a, psum_out.at[cid, pl.ds(r, CHUNK)])


@functools.partial(
    pl.kernel,
    out_type=jax.ShapeDtypeStruct((NW, NP), jnp.float32),
    mesh=_MESH,
    compiler_params=_SC_PARAMS,
    scratch_types=[
        pltpu.VMEM((CH, CHUNK), jnp.int32),
        pltpu.VMEM((CH, CHUNK), jnp.int32),
        pltpu.VMEM((N,), jnp.float32),
        pltpu.VMEM((NP,), jnp.float32),
    ],
)
def _sc_prop1(src3, dst3, t_hbm, psum_out, src_v, dst_v, t_v, acc_v):
    cid = lax.axis_index("c")
    sid = lax.axis_index("s")
    w = cid * NS + sid
    pltpu.sync_copy(t_hbm, t_v)
    pltpu.sync_copy(src3.at[pl.ds(w * CH, CH)], src_v)
    pltpu.sync_copy(dst3.at[pl.ds(w * CH, CH)], dst_v)
    _zero_1d(acc_v, NP)

    @pl.loop(0, CH)
    def _(j):
        for k in range(CHUNK // 16):
            sidx = src_v[j, pl.ds(k * 16, 16)]
            didx = dst_v[j, pl.ds(k * 16, 16)]
            vals = plsc.load_gather(t_v, [sidx])
            plsc.addupdate_scatter(acc_v, [didx], vals)

    pltpu.sync_copy(acc_v, psum_out.at[w])


def _tc_a_body(ct_ref, x_ref, w0_ref, dinv_ref, h0s_ref):
    ct = ct_ref[...]
    deg = jnp.sum(ct, axis=0, keepdims=True) + 1.0
    dinv = jnp.transpose(lax.rsqrt(deg))
    dinv_ref[...] = dinv
    h0 = jnp.dot(x_ref[...], w0_ref[...], preferred_element_type=jnp.float32)
    h0s_ref[...] = h0 * dinv[:N]


def _tc_mid_body(p_ref, hs_ref, dinv_ref, b_ref, g_ref, beta_ref, w_ref,
                 out_ref):
    dinv = dinv_ref[...][:N]
    p = p_ref[...]
    z = dinv * (p[0, :N] + p[1, :N] + hs_ref[...]) + b_ref[...]
    mu = jnp.mean(z, axis=0, keepdims=True)
    zc = z - mu
    var = jnp.mean(zc * zc, axis=0, keepdims=True)
    zn = g_ref[...] * zc * lax.rsqrt(var + 1e-5) + beta_ref[...]
    a = jnp.maximum(zn, 0.0)
    out_ref[...] = (
        jnp.dot(a, w_ref[...], preferred_element_type=jnp.float32) * dinv
    )


def _tc_d_body(p2_ref, h2s_ref, dinv_ref, b2_ref, out_ref):
    dinv = dinv_ref[...][:N]
    p2 = p2_ref[...]
    psum = jnp.transpose(jnp.sum(p2, axis=0, keepdims=True))[:N]
    z = dinv * (psum + h2s_ref[...]) + b2_ref[...]
    out_ref[...] = jax.nn.sigmoid(z)


def kernel(x, edge_index, W0, b0, W1, b1, W2, b2, g0, beta0, g1, beta1):
    ei = edge_index.astype(jnp.int32)
    pad = EPAD - E
    src3 = jnp.concatenate([ei[0], jnp.zeros((pad,), jnp.int32)])
    src3 = src3.reshape(NW * CH, CHUNK)
    dst3 = jnp.concatenate([ei[1], jnp.full((pad,), N, jnp.int32)])
    dst3 = dst3.reshape(NW * CH, CHUNK)

    counts = _sc_degree(dst3)

    dinv_col, h0s = pl.pallas_call(
        _tc_a_body,
        out_shape=[
            jax.ShapeDtypeStruct((NP, 1), jnp.float32),
            jax.ShapeDtypeStruct((N, 32), jnp.float32),
        ],
    )(counts, x, W0)

    p0 = _sc_prop32(src3, dst3, h0s)
    h1s = pl.pallas_call(
        _tc_mid_body,
        out_shape=jax.ShapeDtypeStruct((N, 32), jnp.float32),
    )(p0, h0s, dinv_col, b0.reshape(1, 32), g0.reshape(1, 32),
      beta0.reshape(1, 32), W1)

    p1 = _sc_prop32(src3, dst3, h1s)
    h2s = pl.pallas_call(
        _tc_mid_body,
        out_shape=jax.ShapeDtypeStruct((N, 1), jnp.float32),
    )(p1, h1s, dinv_col, b1.reshape(1, 32), g1.reshape(1, 32),
      beta1.reshape(1, 32), W2)

    p2 = _sc_prop1(src3, dst3, h2s.reshape(N))
    out = pl.pallas_call(
        _tc_d_body,
        out_shape=jax.ShapeDtypeStruct((N, 1), jnp.float32),
    )(p2, h2s, dinv_col, b2.reshape(1, 1))
    return out
